# Initial kernel scaffold; baseline (speedup 1.0000x reference)
import jax
import jax.numpy as jnp
from jax.experimental import pallas as pl


def kernel(x, table, W, b):
    raise NotImplementedError("write your pallas kernel here")


# speedup vs baseline: 6.2171x; 6.2171x over previous
import functools

import jax
import jax.numpy as jnp
from jax import lax
from jax.experimental import pallas as pl
from jax.experimental.pallas import tpu as pltpu
from jax.experimental.pallas import tpu_sc as plsc

NC = 2
NS = 16
NW = NC * NS
L = 16


def _make_sc_kernel(n: int, ch: int):
    per_w = n // NW
    n_chunks = per_w // ch
    mesh = plsc.VectorSubcoreMesh(core_axis_name="c", subcore_axis_name="s")

    @functools.partial(
        pl.kernel,
        out_type=jax.ShapeDtypeStruct((2 * n,), jnp.float32),
        mesh=mesh,
        compiler_params=pltpu.CompilerParams(needs_layout_passes=False),
        scratch_types=[
            pltpu.VMEM((ch,), jnp.int32),
            pltpu.VMEM((2 * ch,), jnp.float32),
            pltpu.VMEM((256,), jnp.float32),
            pltpu.VMEM((256,), jnp.float32),
            pltpu.VMEM((16,), jnp.float32),
        ],
    )
    def sc_kernel(x_hbm, tab_hbm, w_hbm, b_hbm, out_hbm,
                  x_v, out_v, tab_v, w_v, b_v):
        cid = lax.axis_index("c")
        sid = lax.axis_index("s")
        wid = sid * NC + cid

        pltpu.sync_copy(tab_hbm, tab_v)
        pltpu.sync_copy(w_hbm, w_v)
        pltpu.sync_copy(b_hbm, b_v)

        iota = lax.iota(jnp.int32, L)
        half = iota >> 1
        par = iota & 1

        lsv = []
        for srow in range(2):
            for v in range(2):
                acc = jnp.zeros((L,), jnp.float32)
                for k in range(8):
                    tt = tab_v[pl.ds(srow * 128 + k * 16, L)]
                    ww = plsc.load_gather(w_v, [k * 32 + 2 * iota + v])
                    acc = acc + tt * ww
                lsv.append(jnp.sum(acc))
        bpat = plsc.load_gather(b_v, [par])
        even = par == 0
        p0 = jnp.where(even, lsv[0], lsv[1]) + bpat
        p1 = jnp.where(even, lsv[2], lsv[3]) + bpat

        def chunk_body(ci, carry):
            base = wid * per_w + ci * ch
            pltpu.sync_copy(x_hbm.at[pl.ds(base, ch)], x_v)

            def vec_body(i, c2):
                pos = i * L + half
                xx_lo = plsc.load_gather(x_v, [pos])
                xx_hi = plsc.load_gather(x_v, [pos + 8])
                o_lo = jnp.where(xx_lo == 0, p0, p1)
                o_hi = jnp.where(xx_hi == 0, p0, p1)
                out_v[pl.ds(i * 2 * L, L)] = o_lo
                out_v[pl.ds(i * 2 * L + L, L)] = o_hi
                return c2

            lax.fori_loop(0, ch // L, vec_body, 0)
            pltpu.sync_copy(out_v, out_hbm.at[pl.ds(2 * base, 2 * ch)])
            return carry

        lax.fori_loop(0, n_chunks, chunk_body, 0)

    return sc_kernel


def kernel(x, table, W, b):
    batch, hist = x.shape
    n = batch * hist
    ch = 10240
    assert n % NW == 0 and (n // NW) % ch == 0
    x_flat = x.reshape(n)
    tab_flat = table.reshape(256)
    w_flat = W.reshape(256)
    b_pad = jnp.zeros((16,), jnp.float32).at[:2].set(b)
    out = _make_sc_kernel(n, ch)(x_flat, tab_flat, w_flat, b_pad)
    return out.reshape(batch, hist, 2)


# speedup vs baseline: 124.4817x; 20.0223x over previous
import functools

import jax
import jax.numpy as jnp
from jax import lax
from jax.experimental import pallas as pl
from jax.experimental.pallas import tpu as pltpu
from jax.experimental.pallas import tpu_sc as plsc

NC = 2
NS = 16
NW = NC * NS
L = 16

JB = 25
IB = 128
JB_CHUNK = 5
IB_PER_W = IB // NW


def _make_sc_kernel():
    mesh = plsc.VectorSubcoreMesh(core_axis_name="c", subcore_axis_name="s")

    @functools.partial(
        pl.kernel,
        out_type=jax.ShapeDtypeStruct((200, IB, 256), jnp.float32),
        mesh=mesh,
        compiler_params=pltpu.CompilerParams(needs_layout_passes=False),
        scratch_types=[
            pltpu.VMEM((JB_CHUNK, IB_PER_W, 1024), jnp.int32),
            pltpu.VMEM((8 * JB_CHUNK, IB_PER_W, 256), jnp.float32),
            pltpu.VMEM((256,), jnp.float32),
            pltpu.VMEM((256,), jnp.float32),
            pltpu.VMEM((16,), jnp.float32),
        ],
    )
    def sc_kernel(x_hbm, tab_hbm, w_hbm, b_hbm, out_hbm,
                  x_v, out_v, tab_v, w_v, b_v):
        cid = lax.axis_index("c")
        sid = lax.axis_index("s")
        wid = sid * NC + cid
        ib0 = wid * IB_PER_W

        pltpu.sync_copy(tab_hbm, tab_v)
        pltpu.sync_copy(w_hbm, w_v)
        pltpu.sync_copy(b_hbm, b_v)

        iota = lax.iota(jnp.int32, L)

        lsv = []
        for srow in range(2):
            for v in range(2):
                acc = jnp.zeros((L,), jnp.float32)
                for k in range(8):
                    tt = tab_v[pl.ds(srow * 128 + k * 16, L)]
                    ww = w_v[pl.ds(v * 128 + k * 16, L)]
                    acc = acc + tt * ww
                lsv.append(jnp.sum(acc))
        bv = b_v[...]
        zf = jnp.zeros((L,), jnp.float32)
        b0 = jnp.sum(jnp.where(iota == 0, bv, zf))
        b1 = jnp.sum(jnp.where(iota == 1, bv, zf))
        l00 = lsv[0] + b0
        l01 = lsv[1] + b1
        l10 = lsv[2] + b0
        l11 = lsv[3] + b1

        def chunk_body(c, carry):
            jb0 = c * JB_CHUNK
            pltpu.sync_copy(
                x_hbm.at[pl.ds(jb0, JB_CHUNK), pl.ds(ib0, IB_PER_W), :], x_v)
            for jl in range(JB_CHUNK):
                for il in range(IB_PER_W):

                    def vec_body(t, c2, jl=jl, il=il):
                        xv = x_v[jl, il, pl.ds(t * L, L)]
                        m = xv == 0
                        o0 = jnp.where(m, l00, l10)
                        o1 = jnp.where(m, l01, l11)
                        jrow = jl * 8 + (t >> 3)
                        lbase = (t & 7) * L
                        out_v[jrow, il, pl.ds(lbase, L)] = o0
                        out_v[jrow, il, pl.ds(128 + lbase, L)] = o1
                        return c2

                    lax.fori_loop(0, 64, vec_body, 0)
            pltpu.sync_copy(
                out_v,
                out_hbm.at[pl.ds(jb0 * 8, 8 * JB_CHUNK), pl.ds(ib0, IB_PER_W), :])
            return carry

        lax.fori_loop(0, JB // JB_CHUNK, chunk_body, 0)

    return sc_kernel


def kernel(x, table, W, b):
    batch, hist = x.shape
    assert batch == 16384 and hist == 200
    x4 = (x.T.reshape(JB, 8, IB, 128)
          .transpose(0, 2, 1, 3)
          .reshape(JB, IB, 1024))
    wt = W.T.reshape(256)
    tab = table.reshape(256)
    b_pad = jnp.zeros((16,), jnp.float32).at[:2].set(b)
    out3 = _make_sc_kernel()(x4, tab, wt, b_pad)
    return (out3.reshape(200, IB, 2, 128)
            .transpose(1, 3, 0, 2)
            .reshape(batch, hist, 2))


# speedup vs baseline: 289.1218x; 2.3226x over previous
import functools

import jax
import jax.numpy as jnp
from jax import lax
from jax.experimental import pallas as pl
from jax.experimental.pallas import tpu as pltpu
from jax.experimental.pallas import tpu_sc as plsc

NC = 2
NS = 16
NW = NC * NS
L = 16

JB = 25
IB = 128
JB_CHUNK = 5
IB_PER_W = IB // NW


def _make_sc_kernel():
    mesh = plsc.VectorSubcoreMesh(core_axis_name="c", subcore_axis_name="s")

    @functools.partial(
        pl.kernel,
        out_type=jax.ShapeDtypeStruct((200, IB, 2, 128), jnp.float32),
        mesh=mesh,
        compiler_params=pltpu.CompilerParams(needs_layout_passes=False),
        scratch_types=[
            pltpu.VMEM((JB_CHUNK, IB_PER_W, 8, 128), jnp.int32),
            pltpu.VMEM((8 * JB_CHUNK, IB_PER_W, 2, 128), jnp.float32),
            pltpu.VMEM((256,), jnp.float32),
            pltpu.VMEM((256,), jnp.float32),
            pltpu.VMEM((16,), jnp.float32),
        ],
    )
    def sc_kernel(x_hbm, tab_hbm, w_hbm, b_hbm, out_hbm,
                  x_v, out_v, tab_v, w_v, b_v):
        cid = lax.axis_index("c")
        sid = lax.axis_index("s")
        wid = sid * NC + cid
        ib0 = wid * IB_PER_W

        pltpu.sync_copy(tab_hbm, tab_v)
        pltpu.sync_copy(w_hbm, w_v)
        pltpu.sync_copy(b_hbm, b_v)

        iota = lax.iota(jnp.int32, L)

        lsv = []
        for srow in range(2):
            for v in range(2):
                acc = jnp.zeros((L,), jnp.float32)
                for k in range(8):
                    tt = tab_v[pl.ds(srow * 128 + k * 16, L)]
                    ww = w_v[pl.ds(v * 128 + k * 16, L)]
                    acc = acc + tt * ww
                lsv.append(jnp.sum(acc))
        bv = b_v[...]
        zf = jnp.zeros((L,), jnp.float32)
        b0 = jnp.sum(jnp.where(iota == 0, bv, zf))
        b1 = jnp.sum(jnp.where(iota == 1, bv, zf))
        l00 = lsv[0] + b0
        l01 = lsv[1] + b1
        l10 = lsv[2] + b0
        l11 = lsv[3] + b1

        def chunk_body(c, carry):
            jb0 = c * JB_CHUNK
            pltpu.sync_copy(
                x_hbm.at[pl.ds(jb0, JB_CHUNK), pl.ds(ib0, IB_PER_W), :, :], x_v)
            for jl in range(JB_CHUNK):
                for il in range(IB_PER_W):

                    @plsc.parallel_loop(0, 64, unroll=4)
                    def vec_body(t, jl=jl, il=il):
                        js = t >> 3
                        lbase = (t & 7) * L
                        xv = x_v[jl, il, js, pl.ds(lbase, L)]
                        m = xv == 0
                        o0 = jnp.where(m, l00, l10)
                        o1 = jnp.where(m, l01, l11)
                        jrow = jl * 8 + js
                        out_v[jrow, il, 0, pl.ds(lbase, L)] = o0
                        out_v[jrow, il, 1, pl.ds(lbase, L)] = o1

            pltpu.sync_copy(
                out_v,
                out_hbm.at[pl.ds(jb0 * 8, 8 * JB_CHUNK),
                           pl.ds(ib0, IB_PER_W), :, :])
            return carry

        lax.fori_loop(0, JB // JB_CHUNK, chunk_body, 0)

    return sc_kernel


def kernel(x, table, W, b):
    batch, hist = x.shape
    assert batch == 16384 and hist == 200
    x4 = x.T.reshape(JB, 8, IB, 128).transpose(0, 2, 1, 3)
    wt = W.T.reshape(256)
    tab = table.reshape(256)
    b_pad = jnp.zeros((16,), jnp.float32).at[:2].set(b)
    out4 = _make_sc_kernel()(x4, tab, wt, b_pad)
    return out4.transpose(1, 3, 0, 2).reshape(batch, hist, 2)
